# Initial kernel scaffold; baseline (speedup 1.0000x reference)
import jax
import jax.numpy as jnp
from jax.experimental import pallas as pl


def kernel(indices, pos_enc):
    raise NotImplementedError("write your pallas kernel here")


# speedup vs baseline: 8.9455x; 8.9455x over previous
import functools

import jax
import jax.numpy as jnp
from jax import lax
from jax.experimental import pallas as pl
from jax.experimental.pallas import tpu as pltpu
from jax.experimental.pallas import tpu_sc as plsc

D_MODEL = 128
_IDXW = 128
_K = 4
_CHUNK = _K * _IDXW


@functools.cache
def _build(B):
    info = plsc.get_sparse_core_info()
    NC, NS = info.num_cores, info.num_subcores
    NW = NC * NS
    per_w = B // NW
    assert B % NW == 0 and per_w % _CHUNK == 0
    n_chunks = per_w // _CHUNK
    idx_rows_per_w = per_w // _IDXW

    mesh = plsc.VectorSubcoreMesh(core_axis_name="c", subcore_axis_name="s")

    @functools.partial(
        pl.kernel,
        mesh=mesh,
        out_type=jax.ShapeDtypeStruct((B, D_MODEL), jnp.float32),
        scratch_types=[
            pltpu.VMEM((_K, _IDXW), jnp.int32),
            pltpu.VMEM((_CHUNK, D_MODEL), jnp.float32),
            pltpu.SemaphoreType.DMA,
        ],
    )
    def gather_kernel(idx_hbm, table_hbm, out_hbm, idx_v, rows_v, sem):
        wid = lax.axis_index("s") * NC + lax.axis_index("c")
        idx_row0 = wid * idx_rows_per_w
        out0 = wid * per_w

        def body(i, carry):
            pltpu.sync_copy(idx_hbm.at[pl.ds(idx_row0 + i * _K, _K)], idx_v)
            copies = [
                pltpu.async_copy(
                    table_hbm.at[idx_v.at[j]],
                    rows_v.at[pl.ds(j * _IDXW, _IDXW)],
                    sem,
                )
                for j in range(_K)
            ]
            for c in copies:
                c.wait()
            pltpu.sync_copy(rows_v, out_hbm.at[pl.ds(out0 + i * _CHUNK, _CHUNK)])
            return carry

        lax.fori_loop(0, n_chunks, body, 0)

    return gather_kernel


def kernel(indices, pos_enc):
    b0, b1 = indices.shape
    B = b0 * b1
    idx2d = indices.astype(jnp.int32).reshape(B // _IDXW, _IDXW)
    out = _build(B)(idx2d, pos_enc)
    return out.reshape(b0, b1, D_MODEL)


# speedup vs baseline: 9.1547x; 1.0234x over previous
import functools

import jax
import jax.numpy as jnp
from jax import lax
from jax.experimental import pallas as pl
from jax.experimental.pallas import tpu as pltpu
from jax.experimental.pallas import tpu_sc as plsc

D_MODEL = 128
_IDXW = 128
_K = 2
_CHUNK = _K * _IDXW
_NBUF = 2


@functools.cache
def _build(B):
    info = plsc.get_sparse_core_info()
    NC, NS = info.num_cores, info.num_subcores
    NW = NC * NS
    per_w = B // NW
    assert B % NW == 0 and per_w % (_CHUNK * _NBUF) == 0
    n_chunks = per_w // _CHUNK
    n_outer = n_chunks // _NBUF
    idx_rows_per_w = per_w // _IDXW

    mesh = plsc.VectorSubcoreMesh(core_axis_name="c", subcore_axis_name="s")

    @functools.partial(
        pl.kernel,
        mesh=mesh,
        out_type=jax.ShapeDtypeStruct((B, D_MODEL), jnp.float32),
        scratch_types=[
            pltpu.VMEM((_NBUF, _K, _IDXW), jnp.int32),
            pltpu.VMEM((_NBUF, _CHUNK, D_MODEL), jnp.float32),
            pltpu.SemaphoreType.DMA,
            pltpu.SemaphoreType.DMA,
            pltpu.SemaphoreType.DMA,
            pltpu.SemaphoreType.DMA,
        ],
    )
    def gather_kernel(idx_hbm, table_hbm, out_hbm, idx_v, rows_v, g0, g1, w0, w1):
        wid = lax.axis_index("s") * NC + lax.axis_index("c")
        idx_row0 = wid * idx_rows_per_w
        out0 = wid * per_w
        gsem = (g0, g1)
        wsem = (w0, w1)

        def chunk_step(g, b, wait_write):
            if wait_write:
                pltpu.make_async_copy(
                    rows_v.at[b],
                    out_hbm.at[pl.ds(out0, _CHUNK)],
                    wsem[b],
                ).wait()
            pltpu.sync_copy(
                idx_hbm.at[pl.ds(idx_row0 + g * _K, _K)], idx_v.at[b]
            )
            copies = [
                pltpu.async_copy(
                    table_hbm.at[idx_v.at[b].at[j]],
                    rows_v.at[b].at[pl.ds(j * _IDXW, _IDXW)],
                    gsem[b],
                )
                for j in range(_K)
            ]
            for c in copies:
                c.wait()
            pltpu.async_copy(
                rows_v.at[b],
                out_hbm.at[pl.ds(out0 + g * _CHUNK, _CHUNK)],
                wsem[b],
            )

        for b in range(_NBUF):
            chunk_step(b, b, wait_write=False)

        def body(o, carry):
            for b in range(_NBUF):
                chunk_step(o * _NBUF + b, b, wait_write=True)
            return carry

        lax.fori_loop(1, n_outer, body, 0)

        for b in range(_NBUF):
            pltpu.make_async_copy(
                rows_v.at[b],
                out_hbm.at[pl.ds(out0, _CHUNK)],
                wsem[b],
            ).wait()

    return gather_kernel


def kernel(indices, pos_enc):
    b0, b1 = indices.shape
    B = b0 * b1
    idx2d = indices.astype(jnp.int32).reshape(B // _IDXW, _IDXW)
    out = _build(B)(idx2d, pos_enc)
    return out.reshape(b0, b1, D_MODEL)


# speedup vs baseline: 9.9760x; 1.0897x over previous
import functools

import jax
import jax.numpy as jnp
from jax import lax
from jax.experimental import pallas as pl
from jax.experimental.pallas import tpu as pltpu
from jax.experimental.pallas import tpu_sc as plsc

D_MODEL = 128
_IDXW = 128
_K = 3
_CHUNK = _K * _IDXW
_NBUF = 2


@functools.cache
def _build(B):
    info = plsc.get_sparse_core_info()
    NC, NS = info.num_cores, info.num_subcores
    NW = NC * NS
    per_w = B // NW
    assert B % NW == 0 and per_w % _IDXW == 0
    idx_rows = per_w // _IDXW
    n_full = per_w // _CHUNK
    rem_k = (per_w - n_full * _CHUNK) // _IDXW
    assert n_full >= _NBUF and n_full % _NBUF == 0

    mesh = plsc.VectorSubcoreMesh(core_axis_name="c", subcore_axis_name="s")

    @functools.partial(
        pl.kernel,
        mesh=mesh,
        out_type=jax.ShapeDtypeStruct((B, D_MODEL), jnp.float32),
        scratch_types=[
            pltpu.VMEM((idx_rows, _IDXW), jnp.int32),
            pltpu.VMEM((_NBUF, _CHUNK, D_MODEL), jnp.float32),
            pltpu.SemaphoreType.DMA,
            pltpu.SemaphoreType.DMA,
            pltpu.SemaphoreType.DMA,
            pltpu.SemaphoreType.DMA,
        ],
    )
    def gather_kernel(idx_hbm, table_hbm, out_hbm, idx_v, rows_v, g0, g1, w0, w1):
        wid = lax.axis_index("s") * NC + lax.axis_index("c")
        out0 = wid * per_w
        gsem = (g0, g1)
        wsem = (w0, w1)

        pltpu.sync_copy(idx_hbm.at[pl.ds(wid * idx_rows, idx_rows)], idx_v)

        def chunk_step(g, b, wait_write, k=_K):
            if wait_write:
                pltpu.make_async_copy(
                    rows_v.at[b],
                    out_hbm.at[pl.ds(out0, _CHUNK)],
                    wsem[b],
                ).wait()
            copies = [
                pltpu.async_copy(
                    table_hbm.at[idx_v.at[g * _K + j]],
                    rows_v.at[b].at[pl.ds(j * _IDXW, _IDXW)],
                    gsem[b],
                )
                for j in range(k)
            ]
            for c in copies:
                c.wait()
            pltpu.async_copy(
                rows_v.at[b].at[pl.ds(0, k * _IDXW)],
                out_hbm.at[pl.ds(out0 + g * _CHUNK, k * _IDXW)],
                wsem[b],
            )

        for b in range(_NBUF):
            chunk_step(b, b, wait_write=False)

        def body(o, carry):
            for b in range(_NBUF):
                chunk_step(o * _NBUF + b, b, wait_write=True)
            return carry

        lax.fori_loop(1, n_full // _NBUF, body, 0)

        if rem_k:
            chunk_step(n_full, n_full % _NBUF, wait_write=True, k=rem_k)

        for b in range(_NBUF):
            last_k = rem_k if (rem_k and b == n_full % _NBUF) else _K
            pltpu.make_async_copy(
                rows_v.at[b].at[pl.ds(0, last_k * _IDXW)],
                out_hbm.at[pl.ds(out0, last_k * _IDXW)],
                wsem[b],
            ).wait()

    return gather_kernel


def kernel(indices, pos_enc):
    b0, b1 = indices.shape
    B = b0 * b1
    idx2d = indices.astype(jnp.int32).reshape(B // _IDXW, _IDXW)
    out = _build(B)(idx2d, pos_enc)
    return out.reshape(b0, b1, D_MODEL)


# speedup vs baseline: 16.1938x; 1.6233x over previous
import functools

import jax
import jax.numpy as jnp
from jax import lax
from jax.experimental import pallas as pl
from jax.experimental.pallas import tpu as pltpu
from jax.experimental.pallas import tpu_sc as plsc

D_MODEL = 128
_IDXW = 128
_CHUNK = _IDXW
_NBUF = 2
_TABLE_ROWS = 8192
_STAGE = 128


@functools.cache
def _build(B):
    info = plsc.get_sparse_core_info()
    NC, NS = info.num_cores, info.num_subcores
    NW = NC * NS
    per_w = B // NW
    assert B % NW == 0 and per_w % (_CHUNK * _NBUF) == 0
    idx_rows = per_w // _IDXW
    n_chunks = per_w // _CHUNK
    rows_per_tile = _TABLE_ROWS // NS
    assert rows_per_tile % _STAGE == 0

    mesh = plsc.VectorSubcoreMesh(core_axis_name="c", subcore_axis_name="s")

    @functools.partial(
        pl.kernel,
        mesh=mesh,
        out_type=jax.ShapeDtypeStruct((B, D_MODEL), jnp.float32),
        scratch_types=[
            pltpu.VMEM((idx_rows, _IDXW), jnp.int32),
            pltpu.VMEM((_NBUF, _CHUNK, D_MODEL), jnp.float32),
            pltpu.MemorySpace.VMEM_SHARED((_TABLE_ROWS, D_MODEL), jnp.float32),
            pltpu.SemaphoreType.DMA,
            pltpu.SemaphoreType.DMA,
            pltpu.SemaphoreType.DMA,
            pltpu.SemaphoreType.DMA,
        ],
    )
    def gather_kernel(idx_hbm, table_hbm, out_hbm, idx_v, rows_v, table_sh,
                      g0, g1, w0, w1):
        sid = lax.axis_index("s")
        wid = sid * NC + lax.axis_index("c")
        out0 = wid * per_w
        gsem = (g0, g1)
        wsem = (w0, w1)

        for r in range(rows_per_tile // _STAGE):
            t0 = sid * rows_per_tile + r * _STAGE
            pltpu.sync_copy(table_hbm.at[pl.ds(t0, _STAGE)], rows_v.at[0])
            pltpu.sync_copy(rows_v.at[0], table_sh.at[pl.ds(t0, _STAGE)])
        pltpu.sync_copy(idx_hbm.at[pl.ds(wid * idx_rows, idx_rows)], idx_v)
        plsc.subcore_barrier()

        def chunk_step(g, b, wait_write):
            if wait_write:
                pltpu.make_async_copy(
                    rows_v.at[b],
                    out_hbm.at[pl.ds(out0, _CHUNK)],
                    wsem[b],
                ).wait()
            pltpu.async_copy(
                table_sh.at[idx_v.at[g]],
                rows_v.at[b],
                gsem[b],
            ).wait()
            pltpu.async_copy(
                rows_v.at[b],
                out_hbm.at[pl.ds(out0 + g * _CHUNK, _CHUNK)],
                wsem[b],
            )

        for b in range(_NBUF):
            chunk_step(b, b, wait_write=False)

        def body(o, carry):
            for b in range(_NBUF):
                chunk_step(o * _NBUF + b, b, wait_write=True)
            return carry

        lax.fori_loop(1, n_chunks // _NBUF, body, 0)

        for b in range(_NBUF):
            pltpu.make_async_copy(
                rows_v.at[b],
                out_hbm.at[pl.ds(out0, _CHUNK)],
                wsem[b],
            ).wait()

    return gather_kernel


def kernel(indices, pos_enc):
    b0, b1 = indices.shape
    B = b0 * b1
    idx2d = indices.astype(jnp.int32).reshape(B // _IDXW, _IDXW)
    out = _build(B)(idx2d, pos_enc)
    return out.reshape(b0, b1, D_MODEL)
